# baseline (speedup 1.0000x reference)
import functools

import jax
import jax.numpy as jnp
from jax import lax
from jax.experimental import pallas as pl
from jax.experimental.pallas import tpu as pltpu
from jax.experimental.pallas import tpu_sc as plsc

V = 1000
D = 1000
DP = 1024
BT = 1024 * 200
NC, NS = 2, 16
NW = NC * NS
B_PER_W = BT // NW
CHUNK = 64
N_CHUNKS = B_PER_W // CHUNK


def _sc_gather(idx_flat, table):
    mesh = plsc.VectorSubcoreMesh(core_axis_name="c", subcore_axis_name="s")

    @functools.partial(
        pl.kernel,
        mesh=mesh,
        out_type=jax.ShapeDtypeStruct((BT, D), jnp.float32),
        scratch_types=[
            pltpu.VMEM((B_PER_W,), jnp.int32),
            pltpu.VMEM((CHUNK, D), jnp.float32),
            pltpu.SemaphoreType.DMA,
        ],
        compiler_params=pltpu.CompilerParams(use_tc_tiling_on_sc=False),
    )
    def k(idx_hbm, table_hbm, out_hbm, idx_v, rows_v, sem):
        wid = lax.axis_index("s") * NC + lax.axis_index("c")
        base = wid * B_PER_W
        pltpu.sync_copy(idx_hbm.at[pl.ds(base, B_PER_W)], idx_v)

        def body(g, carry):
            start = g * CHUNK
            pltpu.async_copy(
                table_hbm.at[idx_v.at[pl.ds(start, CHUNK)]], rows_v, sem
            ).wait()
            pltpu.sync_copy(rows_v, out_hbm.at[pl.ds(base + start, CHUNK)])
            return carry

        lax.fori_loop(0, N_CHUNKS, body, 0)

    return k(idx_flat, table)


def kernel(idx, token_embedding_table):
    idx_flat = idx.reshape(-1).astype(jnp.int32)
    out = _sc_gather(idx_flat, token_embedding_table)
    return out.reshape(idx.shape[0], idx.shape[1], D)


# speedup vs baseline: 1.0153x; 1.0153x over previous
import functools

import jax
import jax.numpy as jnp
from jax import lax
from jax.experimental import pallas as pl
from jax.experimental.pallas import tpu as pltpu
from jax.experimental.pallas import tpu_sc as plsc

V = 1000
D = 1000
DP = 1024
BT = 1024 * 200
NC, NS = 2, 16
NW = NC * NS
B_PER_W = BT // NW
CHUNK = 40
N_CHUNKS = B_PER_W // CHUNK


def _sc_gather(idx_flat, table):
    mesh = plsc.VectorSubcoreMesh(core_axis_name="c", subcore_axis_name="s")

    @functools.partial(
        pl.kernel,
        mesh=mesh,
        out_type=jax.ShapeDtypeStruct((BT, D), jnp.float32),
        scratch_types=[
            pltpu.VMEM((B_PER_W,), jnp.int32),
            pltpu.VMEM((CHUNK, D), jnp.float32),
            pltpu.VMEM((CHUNK, D), jnp.float32),
            pltpu.SemaphoreType.DMA,
            pltpu.SemaphoreType.DMA,
        ],
        compiler_params=pltpu.CompilerParams(use_tc_tiling_on_sc=False),
    )
    def k(idx_hbm, table_hbm, out_hbm, idx_v, rows0, rows1, sem_g, sem_w):
        wid = lax.axis_index("s") * NC + lax.axis_index("c")
        base = wid * B_PER_W
        pltpu.sync_copy(idx_hbm.at[pl.ds(base, B_PER_W)], idx_v)

        def gather(g, buf):
            pltpu.async_copy(
                table_hbm.at[idx_v.at[pl.ds(g * CHUNK, CHUNK)]], buf, sem_g
            )

        def wr(g, buf):
            pltpu.async_copy(buf, out_hbm.at[pl.ds(base + g * CHUNK, CHUNK)], sem_w)

        def wait_gather():
            pltpu.make_async_copy(
                table_hbm.at[idx_v.at[pl.ds(0, CHUNK)]], rows0, sem_g
            ).wait()

        def wait_write():
            pltpu.make_async_copy(
                rows0, out_hbm.at[pl.ds(base, CHUNK)], sem_w
            ).wait()

        gather(0, rows0)
        wait_gather()
        wr(0, rows0)
        gather(1, rows1)

        def body(h, carry):
            g = 2 * h + 1
            wait_gather()
            wait_write()
            wr(g, rows1)
            gather(g + 1, rows0)
            wait_gather()
            wait_write()
            wr(g + 1, rows0)
            gather(g + 2, rows1)
            return carry

        lax.fori_loop(0, (N_CHUNKS - 2) // 2, body, 0)
        wait_gather()
        wait_write()
        wr(N_CHUNKS - 1, rows1)
        wait_write()

    return k(idx_flat, table)


def kernel(idx, token_embedding_table):
    idx_flat = idx.reshape(-1).astype(jnp.int32)
    out = _sc_gather(idx_flat, token_embedding_table)
    return out.reshape(idx.shape[0], idx.shape[1], D)


# speedup vs baseline: 1.7443x; 1.7179x over previous
import functools

import jax
import jax.numpy as jnp
from jax import lax
from jax.experimental import pallas as pl
from jax.experimental.pallas import tpu as pltpu
from jax.experimental.pallas import tpu_sc as plsc

V = 1000
D = 1000
NT = 7
TAIL = D - NT * 128
BT = 1024 * 200
NC, NS = 2, 16
NW = NC * NS
B_PER_W = BT // NW
CHUNK = 40
N_CHUNKS = B_PER_W // CHUNK
TAIL_COPIES = tuple(
    (128 - TAIL + k * 16, NT * 128 + k * 16) for k in range(TAIL // 16)
)


def _sc_gather(idx_flat, table, table_tail):
    mesh = plsc.VectorSubcoreMesh(core_axis_name="c", subcore_axis_name="s")

    @functools.partial(
        pl.kernel,
        mesh=mesh,
        out_type=jax.ShapeDtypeStruct((BT, D), jnp.float32),
        scratch_types=[
            pltpu.VMEM((B_PER_W,), jnp.int32),
            pltpu.VMEM((CHUNK, D), jnp.float32),
            pltpu.VMEM((CHUNK, D), jnp.float32),
            pltpu.VMEM((CHUNK, 128), jnp.float32),
            pltpu.VMEM((CHUNK, 128), jnp.float32),
            pltpu.SemaphoreType.DMA,
            pltpu.SemaphoreType.DMA,
        ],
        compiler_params=pltpu.CompilerParams(needs_layout_passes=False),
    )
    def k(idx_hbm, table_hbm, tail_hbm, out_hbm,
          idx_v, rows_a, rows_b, last_a, last_b, sem_g, sem_w):
        wid = lax.axis_index("s") * NC + lax.axis_index("c")
        base = wid * B_PER_W
        pltpu.sync_copy(idx_hbm.at[pl.ds(base, B_PER_W)], idx_v)

        def gathers(g, rows, last):
            s = idx_v.at[pl.ds(g * CHUNK, CHUNK)]
            for j in range(NT):
                pltpu.async_copy(
                    table_hbm.at[s, pl.ds(j * 128, 128)],
                    rows.at[:, pl.ds(j * 128, 128)],
                    sem_g,
                )
            pltpu.async_copy(tail_hbm.at[s], last, sem_g)

        def wait_gathers():
            for _ in range(NT + 1):
                pltpu.make_async_copy(
                    tail_hbm.at[idx_v.at[pl.ds(0, CHUNK)]], last_a, sem_g
                ).wait()

        def tailpass(rows, last):
            col_ids = lax.iota(jnp.int32, 16) + (D - 16)

            def rbody(r, carry):
                for src, dst in TAIL_COPIES:
                    rows[r, pl.ds(dst, 16)] = last[r, pl.ds(src, 16)]
                x = last[r, pl.ds(112, 16)]
                row_ids = jnp.full((16,), r, dtype=jnp.int32)
                plsc.store_scatter(rows, [row_ids, col_ids], x)
                return carry

            lax.fori_loop(0, CHUNK, rbody, 0)

        def wr(g, rows):
            pltpu.async_copy(
                rows, out_hbm.at[pl.ds(base + g * CHUNK, CHUNK)], sem_w
            )

        def wait_write():
            pltpu.make_async_copy(
                rows_a, out_hbm.at[pl.ds(base, CHUNK)], sem_w
            ).wait()

        gathers(0, rows_a, last_a)
        wait_gathers()
        tailpass(rows_a, last_a)
        wr(0, rows_a)
        gathers(1, rows_b, last_b)

        def body(h, carry):
            g = 2 * h + 1
            wait_gathers()
            wait_write()
            gathers(g + 1, rows_a, last_a)
            tailpass(rows_b, last_b)
            wr(g, rows_b)
            wait_gathers()
            wait_write()
            gathers(g + 2, rows_b, last_b)
            tailpass(rows_a, last_a)
            wr(g + 1, rows_a)
            return carry

        lax.fori_loop(0, (N_CHUNKS - 2) // 2, body, 0)
        wait_gathers()
        wait_write()
        tailpass(rows_b, last_b)
        wr(N_CHUNKS - 1, rows_b)
        wait_write()

    return k(idx_flat, table, table_tail)


def kernel(idx, token_embedding_table):
    idx_flat = idx.reshape(-1).astype(jnp.int32)
    table_tail = token_embedding_table[:, D - 128:]
    out = _sc_gather(idx_flat, token_embedding_table, table_tail)
    return out.reshape(idx.shape[0], idx.shape[1], D)
